# baseline (speedup 1.0000x reference)
import jax
import jax.numpy as jnp
from jax import lax
from jax.experimental import pallas as pl
from jax.experimental.pallas import tpu as pltpu
from jax.experimental.pallas import tpu_sc as plsc

B = 4
N = 400000
CUBE = 128
ELEMS_PER_B = CUBE * CUBE * CUBE * 4
NFLOAT = B * ELEMS_PER_B
PAD = 32 * 16 * 16
NFLOATP = NFLOAT + PAD
NTILES = 32
P2 = (B * N) // NTILES
GP = 64
CH = 2048
NCH_FULL = P2 // CH
TAIL = P2 - NCH_FULL * CH
TAIL_FULL_GROUPS = TAIL // GP
TAIL_REM_VECS = (TAIL - TAIL_FULL_GROUPS * GP) // 16
MAGIC = float(1.5 * 2 ** 23)


def _sc_body(pts0, pts1, cons, vals, out, ptsbuf, consbuf, valbuf, idxbuf):
    c = lax.axis_index("c")
    s = lax.axis_index("s")
    w = s * 2 + c
    b = w // 8
    pltpu.sync_copy(cons.at[pl.ds(pl.multiple_of(b * 96, 8), 96)], consbuf)
    hx = consbuf[pl.ds(0, 16)]
    hy = consbuf[pl.ds(16, 16)]
    hz = consbuf[pl.ds(32, 16)]
    ivx = consbuf[pl.ds(48, 16)]
    ivy = consbuf[pl.ds(64, 16)]
    ivz = consbuf[pl.ds(80, 16)]
    iota = lax.iota(jnp.int32, 16)
    iota3 = iota * 3
    trash = NFLOAT + w * 256 + iota * 16
    base_pts = w * (P2 * 3)

    for t, pts in ((0, pts0), (1, pts1)):
        pltpu.sync_copy(vals.at[pl.ds(t * 128, 128)], valbuf)
        ebase = b * ELEMS_PER_B + t * 2

        def emit_vec(gbase3, v):
            idxv = iota3 + (gbase3 + v * 48)
            x = plsc.load_gather(ptsbuf, [idxv])
            y = plsc.load_gather(ptsbuf, [idxv + 1])
            z = plsc.load_gather(ptsbuf, [idxv + 2])
            fx = ((x - hx) * ivx + MAGIC) - MAGIC
            fy = ((y - hy) * ivy + MAGIC) - MAGIC
            fz = ((z - hz) * ivz + MAGIC) - MAGIC
            xi = fx.astype(jnp.int32) + (CUBE // 2)
            yi = fy.astype(jnp.int32) + (CUBE // 2)
            zi = fz.astype(jnp.int32) + (CUBE // 2)
            valid = ((xi >= 0) & (xi < CUBE)
                     & (yi >= 0) & (yi < CUBE)
                     & (zi >= 0) & (zi < CUBE))
            ea = (xi * (CUBE * CUBE * 4) + yi * (CUBE * 4) + zi * 4 + ebase)
            idxbuf[pl.ds(v * 16, 16)] = jnp.where(valid, ea, trash)
            idxbuf[pl.ds(64 + v * 16, 16)] = jnp.where(valid, ea + 1, trash + 1)

        def emit_group(gbase3, nvec):
            for v in range(nvec):
                emit_vec(gbase3, v)
            pltpu.sync_copy(valbuf, out.at[idxbuf])

        def chunk_body(j, _):
            off = pl.multiple_of(base_pts + j * (CH * 3), 8)
            pltpu.sync_copy(pts.at[pl.ds(off, CH * 3)], ptsbuf)

            def group_body(g, _):
                emit_group(g * (GP * 3), GP // 16)
                return 0

            lax.fori_loop(0, CH // GP, group_body, 0)
            return 0

        lax.fori_loop(0, NCH_FULL, chunk_body, 0)

        toff = pl.multiple_of(base_pts + NCH_FULL * (CH * 3), 8)
        pltpu.sync_copy(pts.at[pl.ds(toff, TAIL * 3)], ptsbuf.at[pl.ds(0, TAIL * 3)])

        def tail_group_body(g, _):
            emit_group(g * (GP * 3), GP // 16)
            return 0

        lax.fori_loop(0, TAIL_FULL_GROUPS, tail_group_body, 0)
        emit_group(TAIL_FULL_GROUPS * (GP * 3), TAIL_REM_VECS)


def _make_sc_kernel():
    mesh = plsc.VectorSubcoreMesh(core_axis_name="c", subcore_axis_name="s",
                                  num_cores=2, num_subcores=16)
    return pl.kernel(
        _sc_body,
        mesh=mesh,
        compiler_params=pltpu.CompilerParams(
            needs_layout_passes=False, use_tc_tiling_on_sc=False),
        scratch_types=[
            pltpu.VMEM((CH * 3,), jnp.float32),
            pltpu.VMEM((96,), jnp.float32),
            pltpu.VMEM((128,), jnp.float32),
            pltpu.VMEM((128,), jnp.int32),
        ],
    )


_sc_kernel = _make_sc_kernel()


def kernel(map_points_lane, map_points_crosswalk, neck_voxel_sizes, emb_weight):
    pts0 = map_points_lane.reshape(-1)
    pts1 = map_points_crosswalk.reshape(-1)
    c6 = jnp.concatenate([neck_voxel_sizes * 0.5, 1.0 / neck_voxel_sizes],
                         axis=1)
    cons = jnp.broadcast_to(c6[:, :, None], (B, 6, 16)).reshape(-1)
    cons = jnp.asarray(cons, jnp.float32)
    vals = jnp.broadcast_to(emb_weight[:, :, None], (2, 2, 64)).reshape(-1)
    vals = jnp.asarray(vals, jnp.float32)
    buf = jax.new_ref(jnp.zeros((NFLOATP,), jnp.float32))
    _sc_kernel(pts0, pts1, cons, vals, buf)
    out = jax.freeze(buf)
    return out[:NFLOAT].reshape(B, CUBE, CUBE, CUBE, 4)
